# Initial kernel scaffold; baseline (speedup 1.0000x reference)
import jax
import jax.numpy as jnp
from jax.experimental import pallas as pl


def kernel(x, table, bias):
    raise NotImplementedError("write your pallas kernel here")


# speedup vs baseline: 1.3265x; 1.3265x over previous
import functools

import jax
import jax.numpy as jnp
from jax import lax
from jax.experimental import pallas as pl
from jax.experimental.pallas import tpu as pltpu
from jax.experimental.pallas import tpu_sc as plsc

_BATCH = 16384
_NF = 26
_NC = 2
_NS = 16
_NW = _NC * _NS
_RPW = _BATCH // _NW
_IPW = _RPW * _NF
_MINOR = 128
_NROW = _IPW // _MINOR
_CHUNKS = _RPW // 16


def _sc_body(x_hbm, table_hbm, out_hbm, idx_v, val_v, out_v, sem):
    wid = lax.axis_index("s") * _NC + lax.axis_index("c")

    pltpu.sync_copy(x_hbm.at[wid], idx_v)
    pltpu.async_copy(table_hbm.at[idx_v], val_v, sem).wait()

    lane = lax.iota(jnp.int32, 16) * _NF

    def chunk_body(c, carry):
        base = c * (16 * _NF)
        acc = jnp.zeros((16,), jnp.float32)
        for f in range(_NF):
            acc = acc + plsc.load_gather(val_v, [lane + (base + f)])
        out_v[pl.ds(c * 16, 16)] = acc
        return carry

    lax.fori_loop(0, _CHUNKS, chunk_body, 0)
    pltpu.sync_copy(out_v, out_hbm.at[pl.ds(wid * _RPW, _RPW)])


_sc_call = pl.kernel(
    _sc_body,
    out_type=jax.ShapeDtypeStruct((_BATCH,), jnp.float32),
    mesh=plsc.VectorSubcoreMesh(core_axis_name="c", subcore_axis_name="s"),
    scratch_types=[
        pltpu.VMEM((_IPW,), jnp.int32),
        pltpu.VMEM((_IPW,), jnp.float32),
        pltpu.VMEM((_RPW,), jnp.float32),
        pltpu.SemaphoreType.DMA,
    ],
    compiler_params=pltpu.CompilerParams(needs_layout_passes=False),
)


@jax.jit
def kernel(x, table, bias):
    xr = x.reshape(_NW, _IPW)
    out = _sc_call(xr, table.reshape(-1))
    return out.reshape(-1, 1) + bias
